# baseline (speedup 1.0000x reference)
import functools

import jax
import jax.numpy as jnp
from jax import lax
from jax.experimental import pallas as pl
from jax.experimental.pallas import tpu as pltpu
from jax.experimental.pallas import tpu_sc as plsc

N_CLUSTERS = 100
EMBED_DIM = 64
BATCH = 16384

_NC = 2
_NS = 16
_NW = _NC * _NS
_B_PER_W = BATCH // _NW
_L = 16
_BLK = 128
_NBLK = _B_PER_W // _BLK


def _sc_embedding_gather(ids, table_flat_T):
    mesh = plsc.VectorSubcoreMesh(core_axis_name="c", subcore_axis_name="s")

    @functools.partial(
        pl.kernel,
        mesh=mesh,
        out_type=jax.ShapeDtypeStruct((EMBED_DIM, BATCH), jnp.float32),
        scratch_types=[
            pltpu.VMEM((_B_PER_W,), jnp.int32),
            pltpu.VMEM((N_CLUSTERS * EMBED_DIM,), jnp.float32),
            pltpu.VMEM((EMBED_DIM, _B_PER_W), jnp.float32),
            pltpu.SemaphoreType.DMA,
        ],
        compiler_params=pltpu.CompilerParams(needs_layout_passes=False),
    )
    def k(ids_hbm, table_hbm, out_hbm, idx_v, tab_v, outT_v, sem_w):
        wid = lax.axis_index("s") * _NC + lax.axis_index("c")
        base = wid * _B_PER_W
        pltpu.sync_copy(table_hbm, tab_v)
        pltpu.sync_copy(ids_hbm.at[pl.ds(base, _B_PER_W)], idx_v)

        writes = []
        for blk in range(_NBLK):

            def body(bg, _):
                col = bg * _L
                bvec = idx_v[pl.ds(col, _L)]
                for d in range(EMBED_DIM):
                    ivec = bvec + (d * N_CLUSTERS)
                    outT_v[d, pl.ds(col, _L)] = plsc.load_gather(tab_v, [ivec])
                return _

            lax.fori_loop(blk * (_BLK // _L), (blk + 1) * (_BLK // _L), body, None)
            writes.append(
                pltpu.async_copy(
                    outT_v.at[:, pl.ds(blk * _BLK, _BLK)],
                    out_hbm.at[:, pl.ds(base + blk * _BLK, _BLK)],
                    sem_w,
                )
            )
        for w in writes:
            w.wait()

    return k(ids, table_flat_T)


def kernel(cluster_ids, embedding_weight):
    ids = cluster_ids.astype(jnp.int32)
    table_flat_T = embedding_weight.T.reshape(-1)
    outT = _sc_embedding_gather(ids, table_flat_T)
    return outT.T


# speedup vs baseline: 1.0798x; 1.0798x over previous
import functools

import jax
import jax.numpy as jnp
from jax import lax
from jax.experimental import pallas as pl
from jax.experimental.pallas import tpu as pltpu
from jax.experimental.pallas import tpu_sc as plsc

N_CLUSTERS = 100
EMBED_DIM = 64
BATCH = 16384

_NC = 2
_NS = 16
_NW = _NC * _NS
_B_PER_W = BATCH // _NW
_L = 16
_BLK = 128
_NBLK = _B_PER_W // _BLK


def _sc_embedding_gather(ids, table_flat_T):
    mesh = plsc.VectorSubcoreMesh(core_axis_name="c", subcore_axis_name="s")

    @functools.partial(
        pl.kernel,
        mesh=mesh,
        out_type=jax.ShapeDtypeStruct((EMBED_DIM, BATCH), jnp.float32),
        scratch_types=[
            pltpu.VMEM((_B_PER_W,), jnp.int32),
            pltpu.VMEM((N_CLUSTERS * EMBED_DIM,), jnp.float32),
            pltpu.VMEM((EMBED_DIM, _B_PER_W), jnp.float32),
            pltpu.SemaphoreType.DMA,
        ],
        compiler_params=pltpu.CompilerParams(needs_layout_passes=False),
    )
    def k(ids_hbm, table_hbm, out_hbm, idx_v, tab_v, outT_v, sem_w):
        wid = lax.axis_index("s") * _NC + lax.axis_index("c")
        base = wid * _B_PER_W
        pltpu.sync_copy(table_hbm, tab_v)
        pltpu.sync_copy(ids_hbm.at[pl.ds(base, _B_PER_W)], idx_v)

        writes = []
        for blk in range(_NBLK):

            @plsc.parallel_loop(blk * (_BLK // _L), (blk + 1) * (_BLK // _L))
            def _body(bg):
                col = bg * _L
                bvec = idx_v[pl.ds(col, _L)]
                for d in range(EMBED_DIM):
                    ivec = bvec + (d * N_CLUSTERS)
                    outT_v[d, pl.ds(col, _L)] = plsc.load_gather(tab_v, [ivec])
            writes.append(
                pltpu.async_copy(
                    outT_v.at[:, pl.ds(blk * _BLK, _BLK)],
                    out_hbm.at[:, pl.ds(base + blk * _BLK, _BLK)],
                    sem_w,
                )
            )
        for w in writes:
            w.wait()

    return k(ids, table_flat_T)


def kernel(cluster_ids, embedding_weight):
    ids = cluster_ids.astype(jnp.int32)
    table_flat_T = embedding_weight.T.reshape(-1)
    outT = _sc_embedding_gather(ids, table_flat_T)
    return outT.T


# speedup vs baseline: 1.3498x; 1.2501x over previous
import functools

import jax
import jax.numpy as jnp
from jax import lax
from jax.experimental import pallas as pl
from jax.experimental.pallas import tpu as pltpu
from jax.experimental.pallas import tpu_sc as plsc

N_CLUSTERS = 100
EMBED_DIM = 64
BATCH = 16384

_NC = 2
_NS = 16
_NW = _NC * _NS
_B_PER_W = BATCH // _NW
_L = 16
_BLK = 128
_NBLK = _B_PER_W // _BLK


def _sc_embedding_gather(ids, table_flat_T):
    mesh = plsc.VectorSubcoreMesh(core_axis_name="c", subcore_axis_name="s")

    @functools.partial(
        pl.kernel,
        mesh=mesh,
        out_type=jax.ShapeDtypeStruct((EMBED_DIM, BATCH), jnp.float32),
        scratch_types=[
            pltpu.VMEM((_B_PER_W,), jnp.int32),
            pltpu.VMEM((N_CLUSTERS * EMBED_DIM,), jnp.float32),
            pltpu.VMEM((EMBED_DIM, _B_PER_W), jnp.float32),
            pltpu.SemaphoreType.DMA,
        ],
        compiler_params=pltpu.CompilerParams(needs_layout_passes=False),
    )
    def k(ids_hbm, table_hbm, out_hbm, idx_v, tab_v, outT_v, sem_w):
        wid = lax.axis_index("s") * _NC + lax.axis_index("c")
        base = wid * _B_PER_W
        pltpu.sync_copy(table_hbm, tab_v)
        pltpu.sync_copy(ids_hbm.at[pl.ds(base, _B_PER_W)], idx_v)

        writes = []
        for blk in range(_NBLK):

            @plsc.parallel_loop(0, (_BLK // _L) * (EMBED_DIM // 8))
            def _body(u):
                bgl = u >> 3
                dg = u & 7
                col = blk * _BLK + bgl * _L
                bvec = idx_v[pl.ds(col, _L)]
                dvec = bvec + dg * (8 * N_CLUSTERS)
                for kk in range(8):
                    ivec = dvec + kk * N_CLUSTERS
                    outT_v[dg * 8 + kk, pl.ds(col, _L)] = plsc.load_gather(
                        tab_v, [ivec]
                    )
            writes.append(
                pltpu.async_copy(
                    outT_v.at[:, pl.ds(blk * _BLK, _BLK)],
                    out_hbm.at[:, pl.ds(base + blk * _BLK, _BLK)],
                    sem_w,
                )
            )
        for w in writes:
            w.wait()

    return k(ids, table_flat_T)


def kernel(cluster_ids, embedding_weight):
    ids = cluster_ids.astype(jnp.int32)
    table_flat_T = embedding_weight.T.reshape(-1)
    outT = _sc_embedding_gather(ids, table_flat_T)
    return outT.T


# speedup vs baseline: 1.3601x; 1.0076x over previous
import functools

import jax
import jax.numpy as jnp
from jax import lax
from jax.experimental import pallas as pl
from jax.experimental.pallas import tpu as pltpu
from jax.experimental.pallas import tpu_sc as plsc

N_CLUSTERS = 100
EMBED_DIM = 64
BATCH = 16384

_NC = 2
_NS = 16
_NW = _NC * _NS
_B_PER_W = BATCH // _NW
_L = 16
_BLK = 128
_NBLK = _B_PER_W // _BLK


def _sc_embedding_gather(ids, table_flat_T):
    mesh = plsc.VectorSubcoreMesh(core_axis_name="c", subcore_axis_name="s")

    @functools.partial(
        pl.kernel,
        mesh=mesh,
        out_type=jax.ShapeDtypeStruct((EMBED_DIM, BATCH), jnp.float32),
        scratch_types=[
            pltpu.VMEM((_B_PER_W,), jnp.int32),
            pltpu.VMEM((N_CLUSTERS * EMBED_DIM,), jnp.float32),
            pltpu.VMEM((EMBED_DIM, _B_PER_W), jnp.float32),
            pltpu.SemaphoreType.DMA,
        ],
        compiler_params=pltpu.CompilerParams(
            needs_layout_passes=False,
            skip_device_barrier=True,
            disable_bounds_checks=True,
            disable_semaphore_checks=True,
        ),
    )
    def k(ids_hbm, table_hbm, out_hbm, idx_v, tab_v, outT_v, sem_w):
        wid = lax.axis_index("s") * _NC + lax.axis_index("c")
        base = wid * _B_PER_W
        pltpu.sync_copy(table_hbm, tab_v)
        pltpu.sync_copy(ids_hbm.at[pl.ds(base, _B_PER_W)], idx_v)

        writes = []
        for blk in range(_NBLK):

            @plsc.parallel_loop(0, (_BLK // _L) * (EMBED_DIM // 8))
            def _body(u):
                bgl = u >> 3
                dg = u & 7
                col = blk * _BLK + bgl * _L
                bvec = idx_v[pl.ds(col, _L)]
                dvec = bvec + dg * (8 * N_CLUSTERS)
                for kk in range(8):
                    ivec = dvec + kk * N_CLUSTERS
                    outT_v[dg * 8 + kk, pl.ds(col, _L)] = plsc.load_gather(
                        tab_v, [ivec]
                    )
            writes.append(
                pltpu.async_copy(
                    outT_v.at[:, pl.ds(blk * _BLK, _BLK)],
                    out_hbm.at[:, pl.ds(base + blk * _BLK, _BLK)],
                    sem_w,
                )
            )
        for w in writes:
            w.wait()

    return k(ids, table_flat_T)


def kernel(cluster_ids, embedding_weight):
    ids = cluster_ids.astype(jnp.int32)
    table_flat_T = embedding_weight.T.reshape(-1)
    outT = _sc_embedding_gather(ids, table_flat_T)
    return outT.T


# speedup vs baseline: 1.3709x; 1.0079x over previous
import functools

import jax
import jax.numpy as jnp
from jax import lax
from jax.experimental import pallas as pl
from jax.experimental.pallas import tpu as pltpu
from jax.experimental.pallas import tpu_sc as plsc

N_CLUSTERS = 100
EMBED_DIM = 64
BATCH = 16384

_NC = 2
_NS = 16
_NW = _NC * _NS
_B_PER_W = BATCH // _NW
_L = 16
_BLK = 128
_NBLK = _B_PER_W // _BLK


def _sc_embedding_gather(ids, table_flat_T):
    mesh = plsc.VectorSubcoreMesh(core_axis_name="c", subcore_axis_name="s")

    @functools.partial(
        pl.kernel,
        mesh=mesh,
        out_type=jax.ShapeDtypeStruct((EMBED_DIM, BATCH), jnp.float32),
        scratch_types=[
            pltpu.VMEM((_B_PER_W,), jnp.int32),
            pltpu.VMEM((N_CLUSTERS * EMBED_DIM,), jnp.float32),
            pltpu.VMEM((EMBED_DIM, _B_PER_W), jnp.float32),
            pltpu.SemaphoreType.DMA,
        ],
        compiler_params=pltpu.CompilerParams(
            needs_layout_passes=False,
            skip_device_barrier=True,
            disable_bounds_checks=True,
            disable_semaphore_checks=True,
        ),
    )
    def k(ids_hbm, table_hbm, out_hbm, idx_v, tab_v, outT_v, sem_w):
        wid = lax.axis_index("s") * _NC + lax.axis_index("c")
        base = wid * _B_PER_W
        pltpu.sync_copy(table_hbm, tab_v)
        pltpu.sync_copy(ids_hbm.at[pl.ds(base, _B_PER_W)], idx_v)

        @plsc.parallel_loop(0, (_B_PER_W // _L) * (EMBED_DIM // 8))
        def _body(u):
            bg = u >> 3
            dg = u & 7
            col = bg * _L
            bvec = idx_v[pl.ds(col, _L)]
            dvec = bvec + dg * (8 * N_CLUSTERS)
            for kk in range(8):
                ivec = dvec + kk * N_CLUSTERS
                outT_v[dg * 8 + kk, pl.ds(col, _L)] = plsc.load_gather(
                    tab_v, [ivec]
                )

        pltpu.async_copy(
            outT_v,
            out_hbm.at[:, pl.ds(base, _B_PER_W)],
            sem_w,
        ).wait()

    return k(ids, table_flat_T)


def kernel(cluster_ids, embedding_weight):
    ids = cluster_ids.astype(jnp.int32)
    table_flat_T = embedding_weight.T.reshape(-1)
    outT = _sc_embedding_gather(ids, table_flat_T)
    return outT.T


# speedup vs baseline: 1.3997x; 1.0211x over previous
import functools

import jax
import jax.numpy as jnp
from jax import lax
from jax.experimental import pallas as pl
from jax.experimental.pallas import tpu as pltpu
from jax.experimental.pallas import tpu_sc as plsc

N_CLUSTERS = 100
EMBED_DIM = 64
BATCH = 16384

_NC = 2
_NS = 16
_NW = _NC * _NS
_B_PER_W = BATCH // _NW
_L = 16
_BLK = 128
_NBLK = _B_PER_W // _BLK


def _sc_embedding_gather(ids, table_flat_T):
    mesh = plsc.VectorSubcoreMesh(core_axis_name="c", subcore_axis_name="s")

    @functools.partial(
        pl.kernel,
        mesh=mesh,
        out_type=jax.ShapeDtypeStruct((EMBED_DIM, BATCH), jnp.float32),
        scratch_types=[
            pltpu.VMEM((_B_PER_W,), jnp.int32),
            pltpu.VMEM((N_CLUSTERS * EMBED_DIM,), jnp.float32),
            pltpu.VMEM((EMBED_DIM, _B_PER_W), jnp.float32),
            pltpu.SemaphoreType.DMA,
        ],
        compiler_params=pltpu.CompilerParams(
            needs_layout_passes=False,
            skip_device_barrier=True,
            disable_bounds_checks=True,
            disable_semaphore_checks=True,
        ),
    )
    def k(ids_hbm, table_hbm, out_hbm, idx_v, tab_v, outT_v, sem_w):
        wid = lax.axis_index("s") * _NC + lax.axis_index("c")
        base = wid * _B_PER_W
        tab_cp = pltpu.async_copy(table_hbm, tab_v, sem_w)
        ids_cp = pltpu.async_copy(ids_hbm.at[pl.ds(base, _B_PER_W)], idx_v, sem_w)
        tab_cp.wait()
        ids_cp.wait()

        @plsc.parallel_loop(0, (_B_PER_W // _L) * (EMBED_DIM // 8), unroll=2)
        def _body(u):
            bg = u >> 3
            dg = u & 7
            col = bg * _L
            bvec = idx_v[pl.ds(col, _L)]
            dvec = bvec + dg * (8 * N_CLUSTERS)
            for kk in range(8):
                ivec = dvec + kk * N_CLUSTERS
                outT_v[dg * 8 + kk, pl.ds(col, _L)] = plsc.load_gather(
                    tab_v, [ivec]
                )

        pltpu.async_copy(
            outT_v,
            out_hbm.at[:, pl.ds(base, _B_PER_W)],
            sem_w,
        ).wait()

    return k(ids, table_flat_T)


def kernel(cluster_ids, embedding_weight):
    ids = cluster_ids.astype(jnp.int32)
    table_flat_T = embedding_weight.T.reshape(-1)
    outT = _sc_embedding_gather(ids, table_flat_T)
    return outT.T
